# baseline (speedup 1.0000x reference)
import functools

import jax
import jax.numpy as jnp
from jax import lax
from jax.experimental import pallas as pl
from jax.experimental.pallas import tpu as pltpu

_BN_EPS = 1e-5
_VMEM_LIMIT = 64 * 1024 * 1024


def _p1_body(x_ref, w1_ref, xb_ref, st_ref, *, nb):
    s = jnp.zeros_like(st_ref[0])
    ss = jnp.zeros_like(s)
    for i in range(nb):
        xi = x_ref[i]
        xb_ref[i] = xi.astype(jnp.bfloat16)
        y1 = jnp.dot(w1_ref[...], xi, preferred_element_type=jnp.float32)
        s = s + jnp.sum(y1, axis=1, keepdims=True)
        ss = ss + jnp.sum(y1 * y1, axis=1, keepdims=True)
    st_ref[0] = s
    st_ref[1] = ss


def _p2_body(xb_ref, w12_ref, st1_ref, pp_ref, st_ref, *, nb, inv_n, cout):
    red = jnp.sum(st1_ref[...], axis=0)
    mean1 = red[0] * inv_n
    var1 = jnp.maximum(red[1] * inv_n - mean1 * mean1, 0.0)
    scale1 = pp_ref[0] * lax.rsqrt(var1 + _BN_EPS)
    shift1 = pp_ref[1] - mean1 * scale1
    w1s = (scale1 * w12_ref[:cout]).astype(jnp.bfloat16)
    w2b = w12_ref[cout:].astype(jnp.bfloat16)
    s = jnp.zeros_like(scale1)
    ss = jnp.zeros_like(s)
    for i in range(nb):
        h1 = jnp.maximum(
            jnp.dot(w1s, xb_ref[i], preferred_element_type=jnp.float32)
            + shift1, 0.0)
        y2 = jnp.dot(w2b, h1.astype(jnp.bfloat16),
                     preferred_element_type=jnp.float32)
        s = s + jnp.sum(y2, axis=1, keepdims=True)
        ss = ss + jnp.sum(y2 * y2, axis=1, keepdims=True)
    st_ref[0] = s
    st_ref[1] = ss
    st_ref[2] = scale1
    st_ref[3] = shift1


def _p3_body(xb_ref, w_ref, st2_ref, pp_ref, out_ref, *, nb, inv_n, cout):
    st2 = st2_ref[...]
    red = jnp.sum(st2[:, :2], axis=0)
    mean2 = red[0] * inv_n
    var2 = jnp.maximum(red[1] * inv_n - mean2 * mean2, 0.0)
    scale2 = pp_ref[0] * lax.rsqrt(var2 + _BN_EPS)
    shift2 = pp_ref[1] - mean2 * scale2
    scale1 = st2[0, 2]
    shift1 = st2[0, 3]
    bskip = pp_ref[2]
    w1s = (scale1 * w_ref[:cout]).astype(jnp.bfloat16)
    w2s = (scale2 * w_ref[cout:2 * cout]).astype(jnp.bfloat16)
    wsb = w_ref[2 * cout:].astype(jnp.bfloat16)
    for i in range(nb):
        xi = xb_ref[i]
        h1 = jnp.maximum(
            jnp.dot(w1s, xi, preferred_element_type=jnp.float32) + shift1, 0.0)
        y2 = jnp.dot(w2s, h1.astype(jnp.bfloat16),
                     preferred_element_type=jnp.float32)
        skip = jnp.dot(wsb, xi, preferred_element_type=jnp.float32)
        out_ref[i] = (jnp.maximum(y2 + shift2, 0.0)
                      + skip + bskip).astype(out_ref.dtype)


def kernel(x, w1, b1, w2, b2, ws, bs, gamma, beta):
    B, Cin, L = x.shape
    Cout = w1.shape[0]
    inv_n = 1.0 / (B * L)
    nb = next(d for d in (8, 4, 2, 1) if B % d == 0)
    G = B // nb
    nb2 = next(d for d in (16, 8, 4, 2, 1) if B % d == 0)
    G2 = B // nb2

    wcat = jnp.concatenate([w1, w2, ws], axis=0)
    params = jnp.stack([gamma, beta, bs], axis=0)

    cp = pltpu.CompilerParams(dimension_semantics=("parallel",),
                              vmem_limit_bytes=_VMEM_LIMIT)
    x_spec = pl.BlockSpec((nb, Cin, L), lambda g: (g, 0, 0))
    x2_spec = pl.BlockSpec((nb2, Cin, L), lambda g: (g, 0, 0))

    def rep(shape):
        nd = len(shape)
        return pl.BlockSpec(shape, lambda g, nd=nd: (0,) * nd)

    pp_spec = rep((3, Cout, 1))

    cost1 = pl.CostEstimate(
        flops=2 * Cout * Cin * B * L + 3 * Cout * B * L,
        transcendentals=0,
        bytes_accessed=4 * Cin * B * L + 2 * Cin * B * L + 4 * Cout * Cin)
    xb, st1 = pl.pallas_call(
        functools.partial(_p1_body, nb=nb),
        grid=(G,),
        in_specs=[x_spec, rep((Cout, Cin))],
        out_specs=(x_spec, pl.BlockSpec((None, 2, Cout, 1),
                                        lambda g: (g, 0, 0, 0))),
        out_shape=(jax.ShapeDtypeStruct((B, Cin, L), jnp.bfloat16),
                   jax.ShapeDtypeStruct((G, 2, Cout, 1), jnp.float32)),
        compiler_params=cp,
        cost_estimate=cost1,
    )(x, wcat)

    cost2 = pl.CostEstimate(
        flops=2 * (Cout * Cin + Cout * Cout) * B * L + 5 * Cout * B * L,
        transcendentals=0,
        bytes_accessed=2 * Cin * B * L + 4 * (Cout * Cin + Cout * Cout))
    st2 = pl.pallas_call(
        functools.partial(_p2_body, nb=nb2, inv_n=inv_n, cout=Cout),
        grid=(G2,),
        in_specs=[x2_spec, rep((2 * Cout, Cin)), rep((G, 2, Cout, 1)),
                  pp_spec],
        out_specs=pl.BlockSpec((None, 4, Cout, 1), lambda g: (g, 0, 0, 0)),
        out_shape=jax.ShapeDtypeStruct((G2, 4, Cout, 1), jnp.float32),
        compiler_params=cp,
        cost_estimate=cost2,
    )(xb, wcat, st1, params)

    cost3 = pl.CostEstimate(
        flops=2 * (2 * Cout * Cin + Cout * Cout) * B * L,
        transcendentals=0,
        bytes_accessed=(2 * Cin * B * L + 4 * Cout * B * L
                        + 4 * (2 * Cout * Cin + Cout * Cout)))
    out = pl.pallas_call(
        functools.partial(_p3_body, nb=nb, inv_n=inv_n, cout=Cout),
        grid=(G,),
        in_specs=[x_spec, rep((3 * Cout, Cin)), rep((G2, 4, Cout, 1)),
                  pp_spec],
        out_specs=pl.BlockSpec((nb, Cout, L), lambda g: (g, 0, 0)),
        out_shape=jax.ShapeDtypeStruct((B, Cout, L), x.dtype),
        compiler_params=cp,
        cost_estimate=cost3,
    )(xb, wcat, st2, params)
    return out
